# baseline (device time: 17523 ns/iter reference)
import jax
import jax.numpy as jnp
from jax import lax
from jax.experimental import pallas as pl
from jax.experimental.pallas import tpu as pltpu

N_DEV = 4
B, Sq, Hq, Dh = 2, 128, 4, 64
BLK = 64


def kernel(x, Wq, K_ext, V_ext, Wo):
    skv_loc = K_ext.shape[1]
    d_model = x.shape[-1]
    dq = Hq * Dh

    Kt = jnp.transpose(K_ext, (0, 2, 3, 1))
    Vt = jnp.transpose(V_ext, (0, 2, 3, 1))

    def body(x_ref, wq_ref, k_ref, v_ref, wo_ref, out_ref,
             x_v, wq_v, wo_v, kv_buf, o_buf,
             send_sems, fwd_sems, recv_sems, mirror_sems, out_sems,
             stage_sems):
        my = lax.axis_index("i")

        stage_x = pltpu.make_async_copy(x_ref, x_v, stage_sems.at[0])
        stage_wq = pltpu.make_async_copy(wq_ref, wq_v, stage_sems.at[1])
        stage_wo = pltpu.make_async_copy(wo_ref, wo_v, stage_sems.at[2])
        stage_x.start()
        stage_wq.start()
        stage_wo.start()

        barrier = pltpu.get_barrier_semaphore()

        def _sig(d):
            pl.semaphore_signal(barrier, inc=1, device_id=(d,),
                                device_id_type=pl.DeviceIdType.MESH)

        @pl.when((my == 1) | (my == 3))
        def _():
            _sig(0)

        @pl.when(my == 2)
        def _():
            _sig(1)
            _sig(3)

        @pl.when(my == 0)
        def _():
            pl.semaphore_wait(barrier, 2)

        @pl.when((my == 1) | (my == 3))
        def _():
            pl.semaphore_wait(barrier, 1)

        def chunk_src(c):
            b, s = divmod(c, 2)
            return (k_ref if s == 0 else v_ref).at[b]

        def chunk_dst(c):
            b, s = divmod(c, 2)
            return kv_buf.at[s, b]

        def mk_send(c, d, i):
            return pltpu.make_async_remote_copy(
                src_ref=chunk_src(c), dst_ref=chunk_dst(c),
                send_sem=send_sems.at[i], recv_sem=recv_sems.at[c],
                device_id=(d,), device_id_type=pl.DeviceIdType.MESH)

        sends = [
            mk_send(0, 1, 0),
            mk_send(1, 3, 1),
            mk_send(1, 1, 2),
            mk_send(0, 3, 3),
            mk_send(2, 1, 4),
            mk_send(3, 3, 5),
            mk_send(3, 1, 6),
            mk_send(2, 3, 7),
        ]
        fwd_k = [pltpu.make_async_remote_copy(
            src_ref=chunk_dst(2 * b), dst_ref=chunk_dst(2 * b),
            send_sem=fwd_sems.at[2 * b], recv_sem=recv_sems.at[2 * b],
            device_id=(2,), device_id_type=pl.DeviceIdType.MESH,
        ) for b in range(B)]
        fwd_v = [pltpu.make_async_remote_copy(
            src_ref=chunk_dst(2 * b + 1), dst_ref=chunk_dst(2 * b + 1),
            send_sem=fwd_sems.at[2 * b + 1], recv_sem=recv_sems.at[2 * b + 1],
            device_id=(2,), device_id_type=pl.DeviceIdType.MESH,
        ) for b in range(B)]
        recvs = [pltpu.make_async_remote_copy(
            src_ref=chunk_dst(c), dst_ref=chunk_dst(c),
            send_sem=fwd_sems.at[c], recv_sem=recv_sems.at[c],
            device_id=(0,), device_id_type=pl.DeviceIdType.MESH,
        ) for c in range(4)]
        mirrors = [pltpu.make_async_copy(
            chunk_src(c), chunk_dst(c), mirror_sems.at[c])
            for c in range(4)]
        out_cps = [pltpu.make_async_copy(
            o_buf.at[b], out_ref.at[b], out_sems.at[b]) for b in range(B)]

        @pl.when(my == 0)
        def _():
            for r in sends:
                r.start()
            for m in mirrors:
                m.start()

        stage_x.wait()
        stage_wq.wait()
        stage_wo.wait()
        wq = wq_v[...]
        q_alls = [
            jnp.dot(x_v[b], wq, preferred_element_type=jnp.float32) * 0.125
            for b in range(B)
        ]

        wo = wo_v[...]
        for b in range(B):
            c_k, c_v = 2 * b, 2 * b + 1

            @pl.when(my == 0)
            def _(c_k=c_k, c_v=c_v):
                mirrors[c_k].wait()
                mirrors[c_v].wait()

            @pl.when(my == 1)
            def _(b=b, c_k=c_k, c_v=c_v):
                recvs[c_k].wait_recv()
                fwd_k[b].start()
                recvs[c_v].wait_recv()

            @pl.when(my == 3)
            def _(b=b, c_k=c_k, c_v=c_v):
                recvs[c_v].wait_recv()
                fwd_v[b].start()
                recvs[c_k].wait_recv()

            @pl.when(my == 2)
            def _(c_k=c_k, c_v=c_v):
                recvs[c_k].wait_recv()
                recvs[c_v].wait_recv()

            q_all = q_alls[b]
            ctx_heads = []
            for h in range(Hq):
                kt = kv_buf[0, b, h]
                vt = kv_buf[1, b, h]
                q0 = q_all[0:BLK, h * Dh:(h + 1) * Dh]
                q1 = q_all[BLK:Sq, h * Dh:(h + 1) * Dh]
                s0 = lax.dot_general(
                    q0, kt[:, 0:BLK], (((1,), (0,)), ((), ())),
                    preferred_element_type=jnp.float32)
                s1 = lax.dot_general(
                    q1, kt, (((1,), (0,)), ((), ())),
                    preferred_element_type=jnp.float32)
                w0 = jnp.exp(s0 - jnp.max(s0, axis=1, keepdims=True))
                w0 = w0 / jnp.sum(w0, axis=1, keepdims=True)
                w1 = jnp.exp(s1 - jnp.max(s1, axis=1, keepdims=True))
                w1 = w1 / jnp.sum(w1, axis=1, keepdims=True)
                c0 = lax.dot_general(
                    w0, vt[:, 0:BLK], (((1,), (1,)), ((), ())),
                    preferred_element_type=jnp.float32)
                c1 = lax.dot_general(
                    w1, vt, (((1,), (1,)), ((), ())),
                    preferred_element_type=jnp.float32)
                ctx_heads.append(jnp.concatenate([c0, c1], axis=0))
            ctx = jnp.concatenate(ctx_heads, axis=1)
            o_buf[b] = jnp.dot(ctx, wo,
                               preferred_element_type=jnp.float32)
            out_cps[b].start()

        for cp in out_cps:
            cp.wait()

        @pl.when(my == 0)
        def _():
            for r in sends:
                r.wait_send()

        @pl.when(my == 1)
        def _():
            for f in fwd_k:
                f.wait_send()

        @pl.when(my == 3)
        def _():
            for f in fwd_v:
                f.wait_send()

    return pl.pallas_call(
        body,
        out_shape=jax.ShapeDtypeStruct((B, Sq, d_model), jnp.float32),
        in_specs=[pl.BlockSpec(memory_space=pl.ANY)] * 5,
        out_specs=pl.BlockSpec(memory_space=pl.ANY),
        scratch_shapes=[
            pltpu.VMEM((B, Sq, d_model), jnp.float32),
            pltpu.VMEM((d_model, dq), jnp.float32),
            pltpu.VMEM((dq, d_model), jnp.float32),
            pltpu.VMEM((2, B, Hq, Dh, skv_loc), jnp.float32),
            pltpu.VMEM((B, Sq, d_model), jnp.float32),
            pltpu.SemaphoreType.DMA((8,)),
            pltpu.SemaphoreType.DMA((4,)),
            pltpu.SemaphoreType.DMA((4,)),
            pltpu.SemaphoreType.DMA((4,)),
            pltpu.SemaphoreType.DMA((B,)),
            pltpu.SemaphoreType.DMA((3,)),
        ],
        compiler_params=pltpu.CompilerParams(collective_id=0),
    )(x, Wq, Kt, Vt, Wo)


# device time: 17353 ns/iter; 1.0098x vs baseline; 1.0098x over previous
import jax
import jax.numpy as jnp
from jax import lax
from jax.experimental import pallas as pl
from jax.experimental.pallas import tpu as pltpu

N_DEV = 4
B, Sq, Hq, Dh = 2, 128, 4, 64
BLK = 64


def kernel(x, Wq, K_ext, V_ext, Wo):
    skv_loc = K_ext.shape[1]
    d_model = x.shape[-1]

    Kt = jnp.transpose(K_ext, (0, 2, 3, 1))
    Vt = jnp.transpose(V_ext, (0, 2, 3, 1))

    def body(x_ref, wq_ref, k_ref, v_ref, wo_ref, out_ref,
             kv_buf, send_sems, fwd_sems, recv_sems, mirror_sems):
        my = lax.axis_index("i")

        barrier = pltpu.get_barrier_semaphore()

        def _sig(d):
            pl.semaphore_signal(barrier, inc=1, device_id=(d,),
                                device_id_type=pl.DeviceIdType.MESH)

        @pl.when((my == 1) | (my == 3))
        def _():
            _sig(0)

        @pl.when(my == 2)
        def _():
            _sig(1)
            _sig(3)

        @pl.when(my == 0)
        def _():
            pl.semaphore_wait(barrier, 2)

        @pl.when((my == 1) | (my == 3))
        def _():
            pl.semaphore_wait(barrier, 1)

        def chunk_src(c):
            b, s = divmod(c, 2)
            return (k_ref if s == 0 else v_ref).at[b]

        def chunk_dst(c):
            b, s = divmod(c, 2)
            return kv_buf.at[s, b]

        def mk_send(c, d, i):
            return pltpu.make_async_remote_copy(
                src_ref=chunk_src(c), dst_ref=chunk_dst(c),
                send_sem=send_sems.at[i], recv_sem=recv_sems.at[c],
                device_id=(d,), device_id_type=pl.DeviceIdType.MESH)

        sends = [
            mk_send(0, 1, 0),
            mk_send(1, 3, 1),
            mk_send(1, 1, 2),
            mk_send(0, 3, 3),
            mk_send(2, 1, 4),
            mk_send(3, 3, 5),
            mk_send(3, 1, 6),
            mk_send(2, 3, 7),
        ]
        fwd_k = [pltpu.make_async_remote_copy(
            src_ref=chunk_dst(2 * b), dst_ref=chunk_dst(2 * b),
            send_sem=fwd_sems.at[2 * b], recv_sem=recv_sems.at[2 * b],
            device_id=(2,), device_id_type=pl.DeviceIdType.MESH,
        ) for b in range(B)]
        fwd_v = [pltpu.make_async_remote_copy(
            src_ref=chunk_dst(2 * b + 1), dst_ref=chunk_dst(2 * b + 1),
            send_sem=fwd_sems.at[2 * b + 1], recv_sem=recv_sems.at[2 * b + 1],
            device_id=(2,), device_id_type=pl.DeviceIdType.MESH,
        ) for b in range(B)]
        recvs = [pltpu.make_async_remote_copy(
            src_ref=chunk_dst(c), dst_ref=chunk_dst(c),
            send_sem=fwd_sems.at[c], recv_sem=recv_sems.at[c],
            device_id=(0,), device_id_type=pl.DeviceIdType.MESH,
        ) for c in range(4)]
        mirrors = [pltpu.make_async_copy(
            chunk_src(c), chunk_dst(c), mirror_sems.at[c])
            for c in range(4)]

        @pl.when(my == 0)
        def _():
            for r in sends:
                r.start()
            for m in mirrors:
                m.start()

        wq = wq_ref[...]
        q_alls = [
            jnp.dot(x_ref[b], wq, preferred_element_type=jnp.float32) * 0.125
            for b in range(B)
        ]

        wo = wo_ref[...]
        for b in range(B):
            c_k, c_v = 2 * b, 2 * b + 1

            @pl.when(my == 0)
            def _(c_k=c_k, c_v=c_v):
                mirrors[c_k].wait()
                mirrors[c_v].wait()

            @pl.when(my == 1)
            def _(b=b, c_k=c_k, c_v=c_v):
                recvs[c_k].wait_recv()
                fwd_k[b].start()
                recvs[c_v].wait_recv()

            @pl.when(my == 3)
            def _(b=b, c_k=c_k, c_v=c_v):
                recvs[c_v].wait_recv()
                fwd_v[b].start()
                recvs[c_k].wait_recv()

            @pl.when(my == 2)
            def _(c_k=c_k, c_v=c_v):
                recvs[c_k].wait_recv()
                recvs[c_v].wait_recv()

            q_all = q_alls[b]
            ctx_heads = []
            for h in range(Hq):
                kt = kv_buf[0, b, h]
                vt = kv_buf[1, b, h]
                q0 = q_all[0:BLK, h * Dh:(h + 1) * Dh]
                q1 = q_all[BLK:Sq, h * Dh:(h + 1) * Dh]
                s0 = lax.dot_general(
                    q0, kt[:, 0:BLK], (((1,), (0,)), ((), ())),
                    preferred_element_type=jnp.float32)
                s1 = lax.dot_general(
                    q1, kt, (((1,), (0,)), ((), ())),
                    preferred_element_type=jnp.float32)
                w0 = jnp.exp(s0 - jnp.max(s0, axis=1, keepdims=True))
                w0 = w0 / jnp.sum(w0, axis=1, keepdims=True)
                w1 = jnp.exp(s1 - jnp.max(s1, axis=1, keepdims=True))
                w1 = w1 / jnp.sum(w1, axis=1, keepdims=True)
                c0 = lax.dot_general(
                    w0, vt[:, 0:BLK], (((1,), (1,)), ((), ())),
                    preferred_element_type=jnp.float32)
                c1 = lax.dot_general(
                    w1, vt, (((1,), (1,)), ((), ())),
                    preferred_element_type=jnp.float32)
                ctx_heads.append(jnp.concatenate([c0, c1], axis=0))
            ctx = jnp.concatenate(ctx_heads, axis=1)
            out_ref[b] = jnp.dot(ctx, wo,
                                 preferred_element_type=jnp.float32)

        @pl.when(my == 0)
        def _():
            for r in sends:
                r.wait_send()

        @pl.when(my == 1)
        def _():
            for f in fwd_k:
                f.wait_send()

        @pl.when(my == 3)
        def _():
            for f in fwd_v:
                f.wait_send()

    return pl.pallas_call(
        body,
        out_shape=jax.ShapeDtypeStruct((B, Sq, d_model), jnp.float32),
        in_specs=[pl.BlockSpec(memory_space=pltpu.VMEM)] * 5,
        out_specs=pl.BlockSpec(memory_space=pltpu.VMEM),
        scratch_shapes=[
            pltpu.VMEM((2, B, Hq, Dh, skv_loc), jnp.float32),
            pltpu.SemaphoreType.DMA((8,)),
            pltpu.SemaphoreType.DMA((4,)),
            pltpu.SemaphoreType.DMA((4,)),
            pltpu.SemaphoreType.DMA((4,)),
        ],
        compiler_params=pltpu.CompilerParams(collective_id=0),
    )(x, Wq, Kt, Vt, Wo)


# device time: 13129 ns/iter; 1.3347x vs baseline; 1.3217x over previous
import jax
import jax.numpy as jnp
from jax import lax
from jax.experimental import pallas as pl
from jax.experimental.pallas import tpu as pltpu

N_DEV = 4
B, Sq, Hq, Dh = 2, 128, 4, 64
BLK = 64


def kernel(x, Wq, K_ext, V_ext, Wo):
    skv_loc = K_ext.shape[1]
    d_model = x.shape[-1]
    dq = Hq * Dh

    def body(x_ref, wq_ref, k_ref, v_ref, wo_ref, out_ref,
             kv_buf, send_sems, fwd_sems, recv_sems):
        my = lax.axis_index("i")

        barrier = pltpu.get_barrier_semaphore()

        def _sig(d):
            pl.semaphore_signal(barrier, inc=1, device_id=(d,),
                                device_id_type=pl.DeviceIdType.MESH)

        @pl.when((my == 1) | (my == 3))
        def _():
            _sig(0)

        @pl.when(my == 2)
        def _():
            _sig(1)
            _sig(3)

        @pl.when(my == 0)
        def _():
            pl.semaphore_wait(barrier, 2)

        @pl.when((my == 1) | (my == 3))
        def _():
            pl.semaphore_wait(barrier, 1)

        def chunk_dst(c):
            b, s = divmod(c, 2)
            return kv_buf.at[s, b]

        def mk_send(c, d, i):
            return pltpu.make_async_remote_copy(
                src_ref=chunk_dst(c), dst_ref=chunk_dst(c),
                send_sem=send_sems.at[i], recv_sem=recv_sems.at[c],
                device_id=(d,), device_id_type=pl.DeviceIdType.MESH)

        sends_b = [
            [mk_send(2 * b + 0, 1, 4 * b + 0),
             mk_send(2 * b + 1, 3, 4 * b + 1),
             mk_send(2 * b + 1, 1, 4 * b + 2),
             mk_send(2 * b + 0, 3, 4 * b + 3)]
            for b in range(B)
        ]
        fwd_k = [pltpu.make_async_remote_copy(
            src_ref=chunk_dst(2 * b), dst_ref=chunk_dst(2 * b),
            send_sem=fwd_sems.at[2 * b], recv_sem=recv_sems.at[2 * b],
            device_id=(2,), device_id_type=pl.DeviceIdType.MESH,
        ) for b in range(B)]
        fwd_v = [pltpu.make_async_remote_copy(
            src_ref=chunk_dst(2 * b + 1), dst_ref=chunk_dst(2 * b + 1),
            send_sem=fwd_sems.at[2 * b + 1], recv_sem=recv_sems.at[2 * b + 1],
            device_id=(2,), device_id_type=pl.DeviceIdType.MESH,
        ) for b in range(B)]
        recvs = [pltpu.make_async_remote_copy(
            src_ref=chunk_dst(c), dst_ref=chunk_dst(c),
            send_sem=fwd_sems.at[c], recv_sem=recv_sems.at[c],
            device_id=(0,), device_id_type=pl.DeviceIdType.MESH,
        ) for c in range(4)]

        @pl.when(my == 0)
        def _():
            for b in range(B):
                kv_buf[0, b] = jnp.reshape(
                    k_ref[b], (skv_loc, dq)).astype(jnp.bfloat16)
                kv_buf[1, b] = jnp.reshape(
                    v_ref[b], (skv_loc, dq)).astype(jnp.bfloat16)
                for r in sends_b[b]:
                    r.start()

        wq = wq_ref[...]
        q_alls = [
            jnp.dot(x_ref[b], wq, preferred_element_type=jnp.float32) * 0.125
            for b in range(B)
        ]

        wo = wo_ref[...]
        for b in range(B):
            c_k, c_v = 2 * b, 2 * b + 1

            @pl.when(my == 1)
            def _(b=b, c_k=c_k, c_v=c_v):
                recvs[c_k].wait_recv()
                fwd_k[b].start()
                recvs[c_v].wait_recv()

            @pl.when(my == 3)
            def _(b=b, c_k=c_k, c_v=c_v):
                recvs[c_v].wait_recv()
                fwd_v[b].start()
                recvs[c_k].wait_recv()

            @pl.when(my == 2)
            def _(c_k=c_k, c_v=c_v):
                recvs[c_k].wait_recv()
                recvs[c_v].wait_recv()

            q_all = q_alls[b].astype(jnp.bfloat16)
            ctx_heads = []
            for h in range(Hq):
                k_bh = kv_buf[0, b, :, h * Dh:(h + 1) * Dh]
                v_bh = kv_buf[1, b, :, h * Dh:(h + 1) * Dh]
                q0 = q_all[0:BLK, h * Dh:(h + 1) * Dh]
                q1 = q_all[BLK:Sq, h * Dh:(h + 1) * Dh]
                s0 = lax.dot_general(
                    q0, k_bh[0:BLK, :], (((1,), (1,)), ((), ())),
                    preferred_element_type=jnp.float32)
                s1 = lax.dot_general(
                    q1, k_bh, (((1,), (1,)), ((), ())),
                    preferred_element_type=jnp.float32)
                w0 = jnp.exp(s0 - jnp.max(s0, axis=1, keepdims=True))
                w0 = (w0 / jnp.sum(w0, axis=1, keepdims=True)
                      ).astype(jnp.bfloat16)
                w1 = jnp.exp(s1 - jnp.max(s1, axis=1, keepdims=True))
                w1 = (w1 / jnp.sum(w1, axis=1, keepdims=True)
                      ).astype(jnp.bfloat16)
                c0 = jnp.dot(w0, v_bh[0:BLK, :],
                             preferred_element_type=jnp.float32)
                c1 = jnp.dot(w1, v_bh,
                             preferred_element_type=jnp.float32)
                ctx_heads.append(jnp.concatenate([c0, c1], axis=0))
            ctx = jnp.concatenate(ctx_heads, axis=1)
            out_ref[b] = jnp.dot(ctx, wo,
                                 preferred_element_type=jnp.float32)

        @pl.when(my == 0)
        def _():
            for bs in sends_b:
                for r in bs:
                    r.wait_send()

        @pl.when(my == 1)
        def _():
            for f in fwd_k:
                f.wait_send()

        @pl.when(my == 3)
        def _():
            for f in fwd_v:
                f.wait_send()

    return pl.pallas_call(
        body,
        out_shape=jax.ShapeDtypeStruct((B, Sq, d_model), jnp.float32),
        in_specs=[pl.BlockSpec(memory_space=pltpu.VMEM)] * 5,
        out_specs=pl.BlockSpec(memory_space=pltpu.VMEM),
        scratch_shapes=[
            pltpu.VMEM((2, B, skv_loc, dq), jnp.bfloat16),
            pltpu.SemaphoreType.DMA((8,)),
            pltpu.SemaphoreType.DMA((4,)),
            pltpu.SemaphoreType.DMA((4,)),
        ],
        compiler_params=pltpu.CompilerParams(collective_id=0),
    )(x, Wq, K_ext, V_ext, Wo)


# device time: 10698 ns/iter; 1.6380x vs baseline; 1.2272x over previous
import jax
import jax.numpy as jnp
from jax import lax
from jax.experimental import pallas as pl
from jax.experimental.pallas import tpu as pltpu

N_DEV = 4
B, Sq, Hq, Dh = 2, 128, 4, 64
BLK = 64


def kernel(x, Wq, K_ext, V_ext, Wo):
    skv_loc = K_ext.shape[1]
    d_model = x.shape[-1]
    dq = Hq * Dh

    def body(x_ref, wq_ref, k_ref, v_ref, wo_ref, out_ref,
             kv_buf, k_stage, v_stage, send_sems, fwd_sems, recv_sems,
             stage_sems):
        my = lax.axis_index("i")

        stage_k = pltpu.make_async_copy(k_ref, k_stage, stage_sems.at[0])
        stage_v = pltpu.make_async_copy(v_ref, v_stage, stage_sems.at[1])

        @pl.when(my == 0)
        def _():
            stage_k.start()
            stage_v.start()

        barrier = pltpu.get_barrier_semaphore()

        def _sig(d):
            pl.semaphore_signal(barrier, inc=1, device_id=(d,),
                                device_id_type=pl.DeviceIdType.MESH)

        @pl.when((my == 1) | (my == 3))
        def _():
            _sig(0)

        @pl.when(my == 2)
        def _():
            _sig(1)
            _sig(3)

        @pl.when(my == 0)
        def _():
            pl.semaphore_wait(barrier, 2)

        @pl.when((my == 1) | (my == 3))
        def _():
            pl.semaphore_wait(barrier, 1)

        def chunk_dst(c):
            b, s = divmod(c, 2)
            return kv_buf.at[s, b]

        def mk_send(c, d, i):
            return pltpu.make_async_remote_copy(
                src_ref=chunk_dst(c), dst_ref=chunk_dst(c),
                send_sem=send_sems.at[i], recv_sem=recv_sems.at[c],
                device_id=(d,), device_id_type=pl.DeviceIdType.MESH)

        sends_b = [
            [mk_send(2 * b + 0, 1, 4 * b + 0),
             mk_send(2 * b + 1, 3, 4 * b + 1),
             mk_send(2 * b + 1, 1, 4 * b + 2),
             mk_send(2 * b + 0, 3, 4 * b + 3)]
            for b in range(B)
        ]
        fwd_k = [pltpu.make_async_remote_copy(
            src_ref=chunk_dst(2 * b), dst_ref=chunk_dst(2 * b),
            send_sem=fwd_sems.at[2 * b], recv_sem=recv_sems.at[2 * b],
            device_id=(2,), device_id_type=pl.DeviceIdType.MESH,
        ) for b in range(B)]
        fwd_v = [pltpu.make_async_remote_copy(
            src_ref=chunk_dst(2 * b + 1), dst_ref=chunk_dst(2 * b + 1),
            send_sem=fwd_sems.at[2 * b + 1], recv_sem=recv_sems.at[2 * b + 1],
            device_id=(2,), device_id_type=pl.DeviceIdType.MESH,
        ) for b in range(B)]
        recvs = [pltpu.make_async_remote_copy(
            src_ref=chunk_dst(c), dst_ref=chunk_dst(c),
            send_sem=fwd_sems.at[c], recv_sem=recv_sems.at[c],
            device_id=(0,), device_id_type=pl.DeviceIdType.MESH,
        ) for c in range(4)]

        @pl.when(my == 0)
        def _():
            stage_k.wait()
            stage_v.wait()
            for b in range(B):
                kv_buf[0, b] = jnp.reshape(
                    k_stage[b], (skv_loc, dq)).astype(jnp.bfloat16)
                kv_buf[1, b] = jnp.reshape(
                    v_stage[b], (skv_loc, dq)).astype(jnp.bfloat16)
                for r in sends_b[b]:
                    r.start()

        wq = wq_ref[...]
        q_alls = [
            jnp.dot(x_ref[b], wq, preferred_element_type=jnp.float32) * 0.125
            for b in range(B)
        ]

        wo = wo_ref[...]
        for b in range(B):
            c_k, c_v = 2 * b, 2 * b + 1

            @pl.when(my == 1)
            def _(b=b, c_k=c_k, c_v=c_v):
                recvs[c_k].wait_recv()
                fwd_k[b].start()
                recvs[c_v].wait_recv()

            @pl.when(my == 3)
            def _(b=b, c_k=c_k, c_v=c_v):
                recvs[c_v].wait_recv()
                fwd_v[b].start()
                recvs[c_k].wait_recv()

            @pl.when(my == 2)
            def _(c_k=c_k, c_v=c_v):
                recvs[c_k].wait_recv()
                recvs[c_v].wait_recv()

            q_all = q_alls[b].astype(jnp.bfloat16)
            ctx_heads = []
            for h in range(Hq):
                k_bh = kv_buf[0, b, :, h * Dh:(h + 1) * Dh]
                v_bh = kv_buf[1, b, :, h * Dh:(h + 1) * Dh]
                q0 = q_all[0:BLK, h * Dh:(h + 1) * Dh]
                q1 = q_all[BLK:Sq, h * Dh:(h + 1) * Dh]
                s0 = lax.dot_general(
                    q0, k_bh[0:BLK, :], (((1,), (1,)), ((), ())),
                    preferred_element_type=jnp.float32)
                s1 = lax.dot_general(
                    q1, k_bh, (((1,), (1,)), ((), ())),
                    preferred_element_type=jnp.float32)
                w0 = jnp.exp(s0 - jnp.max(s0, axis=1, keepdims=True))
                w0 = (w0 / jnp.sum(w0, axis=1, keepdims=True)
                      ).astype(jnp.bfloat16)
                w1 = jnp.exp(s1 - jnp.max(s1, axis=1, keepdims=True))
                w1 = (w1 / jnp.sum(w1, axis=1, keepdims=True)
                      ).astype(jnp.bfloat16)
                c0 = jnp.dot(w0, v_bh[0:BLK, :],
                             preferred_element_type=jnp.float32)
                c1 = jnp.dot(w1, v_bh,
                             preferred_element_type=jnp.float32)
                ctx_heads.append(jnp.concatenate([c0, c1], axis=0))
            ctx = jnp.concatenate(ctx_heads, axis=1)
            out_ref[b] = jnp.dot(ctx, wo,
                                 preferred_element_type=jnp.float32)

        @pl.when(my == 0)
        def _():
            for bs in sends_b:
                for r in bs:
                    r.wait_send()

        @pl.when(my == 1)
        def _():
            for f in fwd_k:
                f.wait_send()

        @pl.when(my == 3)
        def _():
            for f in fwd_v:
                f.wait_send()

    return pl.pallas_call(
        body,
        out_shape=jax.ShapeDtypeStruct((B, Sq, d_model), jnp.float32),
        in_specs=[pl.BlockSpec(memory_space=pltpu.VMEM)] * 2
        + [pl.BlockSpec(memory_space=pl.ANY)] * 2
        + [pl.BlockSpec(memory_space=pltpu.VMEM)],
        out_specs=pl.BlockSpec(memory_space=pltpu.VMEM),
        scratch_shapes=[
            pltpu.VMEM((2, B, skv_loc, dq), jnp.bfloat16),
            pltpu.VMEM((B, skv_loc, Hq, Dh), jnp.float32),
            pltpu.VMEM((B, skv_loc, Hq, Dh), jnp.float32),
            pltpu.SemaphoreType.DMA((8,)),
            pltpu.SemaphoreType.DMA((4,)),
            pltpu.SemaphoreType.DMA((4,)),
            pltpu.SemaphoreType.DMA((2,)),
        ],
        compiler_params=pltpu.CompilerParams(collective_id=0),
    )(x, Wq, K_ext, V_ext, Wo)
